# Initial kernel scaffold; baseline (speedup 1.0000x reference)
import functools

import jax
import jax.numpy as jnp
from jax import lax
from jax.experimental import pallas as pl
from jax.experimental.pallas import tpu as pltpu
from jax.experimental.pallas import tpu_sc as plsc

N = 10000
E = 320000
R = 8
NB = 8
D = 128
RN = R * N
NC = 2
NS = 16
NW = NC * NS
EPT = E // NW

KC = 80
CC = EPT // KC
KE = 125
CE = EPT // KE
ROWS16 = RN // 16
RZ = 312
NT = 10
BN = N // NT

f32 = jnp.float32
i32 = jnp.int32

_mesh = plsc.VectorSubcoreMesh(core_axis_name="c", subcore_axis_name="s")


def _wid():
    return lax.axis_index("s") * NC + lax.axis_index("c")


@functools.partial(
    pl.kernel,
    out_type=jax.ShapeDtypeStruct((NC, ROWS16, 16), f32),
    mesh=_mesh,
    scratch_types=[
        pltpu.VMEM((EPT,), i32),
        pltpu.VMEM((EPT,), i32),
        pltpu.VMEM((CC, KC), i32),
        pltpu.VMEM((KC, 16), f32),
        pltpu.VMEM_SHARED((ROWS16, 16), f32),
        pltpu.VMEM((RZ + 16, 16), f32),
        pltpu.SemaphoreType.DMA,
    ],
)
def _sc_count(dst_hbm, et_hbm, out_hbm, dst_v, et_v, rix_v, oh_v, cnt_sh,
              cp_v, sem):
    c = lax.axis_index("c")
    s = lax.axis_index("s")
    wid = _wid()
    base = wid * EPT
    pltpu.async_copy(dst_hbm.at[pl.ds(base, EPT)], dst_v, sem).wait()
    pltpu.async_copy(et_hbm.at[pl.ds(base, EPT)], et_v, sem).wait()

    zrow = jnp.zeros((16,), f32)
    def _z(j, _):
        cp_v[j, :] = zrow
        return 0
    lax.fori_loop(0, RZ + 16, _z, 0)
    pltpu.sync_copy(cp_v.at[pl.ds(0, RZ)], cnt_sh.at[pl.ds(s * RZ, RZ)])
    @pl.when(s == NS - 1)
    def _():
        pltpu.sync_copy(cp_v.at[pl.ds(0, ROWS16 - NS * RZ)],
                        cnt_sh.at[pl.ds(NS * RZ, ROWS16 - NS * RZ)])
    plsc.subcore_barrier()

    lane = lax.iota(i32, 16)
    ones = jnp.ones((16,), f32)

    def _chunk(j, _):
        for k in range(KC // 16):
            off = j * KC + k * 16
            d16 = dst_v[pl.ds(off, 16)]
            t16 = et_v[pl.ds(off, 16)]
            seg = t16 * N + d16
            rix_v[j, pl.ds(k * 16, 16)] = seg >> 4
            for q in range(16):
                oh_v[k * 16 + q, :] = zrow
            plsc.store_scatter(oh_v, [k * 16 + lane, seg & 15], ones)
        pltpu.sync_copy(oh_v, cnt_sh.at[rix_v.at[j]], add=True)
        return 0

    lax.fori_loop(0, CC, _chunk, 0)
    plsc.subcore_barrier()

    pltpu.sync_copy(cnt_sh.at[pl.ds(s * RZ, RZ)], cp_v.at[pl.ds(0, RZ)])
    pltpu.sync_copy(cp_v.at[pl.ds(0, RZ)], out_hbm.at[c, pl.ds(s * RZ, RZ)])
    @pl.when(s == NS - 1)
    def _():
        tail = ROWS16 - NS * RZ
        pltpu.sync_copy(cnt_sh.at[pl.ds(NS * RZ, tail)],
                        cp_v.at[pl.ds(0, tail)])
        pltpu.sync_copy(cp_v.at[pl.ds(0, tail)],
                        out_hbm.at[c, pl.ds(NS * RZ, tail)])


def _cinv_body(p_ref, o_ref):
    o_ref[...] = 1.0 / jnp.maximum(p_ref[0] + p_ref[1], 1.0)


def _tc_cinv(parts):
    p3 = parts.reshape(NC, RN // 128, 128)
    return pl.pallas_call(
        _cinv_body,
        out_shape=jax.ShapeDtypeStruct((RN // 128, 128), f32),
        grid=(5,),
        in_specs=[pl.BlockSpec((NC, RN // 128 // 5, 128),
                               lambda i: (0, i, 0))],
        out_specs=pl.BlockSpec((RN // 128 // 5, 128), lambda i: (i, 0)),
    )(p3)


@functools.partial(
    pl.kernel,
    out_type=[jax.ShapeDtypeStruct((NW, EPT), i32),
              jax.ShapeDtypeStruct((NW, EPT), f32)],
    mesh=_mesh,
    scratch_types=[
        pltpu.VMEM((RN,), f32),
        pltpu.VMEM((EPT,), i32),
        pltpu.VMEM((EPT,), i32),
        pltpu.VMEM((EPT,), i32),
        pltpu.VMEM((EPT,), f32),
        pltpu.SemaphoreType.DMA,
    ],
)
def _sc_prep(src_hbm, dst_hbm, et_hbm, cinv_hbm, gidx_hbm, w_hbm,
             ci_v, s_v, d_v, t_v, w_v, sem):
    wid = _wid()
    base = wid * EPT
    pltpu.async_copy(cinv_hbm, ci_v, sem).wait()
    pltpu.async_copy(src_hbm.at[pl.ds(base, EPT)], s_v, sem).wait()
    pltpu.async_copy(dst_hbm.at[pl.ds(base, EPT)], d_v, sem).wait()
    pltpu.async_copy(et_hbm.at[pl.ds(base, EPT)], t_v, sem).wait()

    def _g(j, _):
        off = j * 16
        s16 = s_v[pl.ds(off, 16)]
        d16 = d_v[pl.ds(off, 16)]
        t16 = t_v[pl.ds(off, 16)]
        seg = t16 * N + d16
        w_v[pl.ds(off, 16)] = plsc.load_gather(ci_v, [seg])
        s_v[pl.ds(off, 16)] = t16 * N + s16
        return 0

    lax.fori_loop(0, EPT // 16, _g, 0)
    pltpu.sync_copy(s_v, gidx_hbm.at[wid])
    pltpu.sync_copy(w_v, w_hbm.at[wid])


@functools.partial(
    pl.kernel,
    out_type=jax.ShapeDtypeStruct((NC, N, D), f32),
    mesh=_mesh,
    scratch_types=[
        pltpu.VMEM((CE, KE), i32),
        pltpu.VMEM((CE, KE), i32),
        pltpu.VMEM((EPT,), f32),
        pltpu.VMEM((KE, D), f32),
        pltpu.VMEM((KE, D), f32),
        pltpu.VMEM((KE, D), f32),
        pltpu.VMEM_SHARED((N, D), f32),
        pltpu.SemaphoreType.DMA,
        pltpu.SemaphoreType.DMA,
        pltpu.SemaphoreType.DMA,
    ],
)
def _sc_edge(y_hbm, gidx_hbm, dst_hbm, w_hbm, out_hbm,
             gi_v, ds_v, w_v, r0_v, r1_v, cp_v, acc_sh, sem0, sem1, semc):
    c = lax.axis_index("c")
    s = lax.axis_index("s")
    wid = _wid()
    pltpu.async_copy(gidx_hbm.at[wid], gi_v, semc).wait()
    pltpu.async_copy(dst_hbm.at[wid], ds_v, semc).wait()
    pltpu.async_copy(w_hbm.at[wid], w_v, semc).wait()

    zrow = jnp.zeros((16,), f32)
    def _z(j, _):
        for q in range(D // 16):
            cp_v[j, pl.ds(q * 16, 16)] = zrow
        return 0
    lax.fori_loop(0, KE, _z, 0)
    for k in range(5):
        pltpu.sync_copy(cp_v, acc_sh.at[pl.ds(s * 625 + k * KE, KE)])
    plsc.subcore_barrier()

    def _scale(slot_ref, j):
        def _e(e, _):
            wv = plsc.load_gather(w_v, [jnp.full((16,), j * KE + e, i32)])
            for q in range(D // 16):
                slot_ref[e, pl.ds(q * 16, 16)] = (
                    slot_ref[e, pl.ds(q * 16, 16)] * wv)
            return 0
        lax.fori_loop(0, KE, _e, 0)

    def _start(slot_ref, sem, j):
        return pltpu.async_copy(y_hbm.at[gi_v.at[j]], slot_ref, sem)

    _start(r0_v, sem0, 0)

    def _pair(jj, _):
        j0 = jj * 2
        _start(r1_v, sem1, j0 + 1)
        pltpu.make_async_copy(y_hbm.at[gi_v.at[j0]], r0_v, sem0).wait()
        _scale(r0_v, j0)
        pltpu.sync_copy(r0_v, acc_sh.at[ds_v.at[j0]], add=True)
        @pl.when(jj < CE // 2 - 1)
        def _():
            _start(r0_v, sem0, j0 + 2)
        pltpu.make_async_copy(y_hbm.at[gi_v.at[j0 + 1]], r1_v, sem1).wait()
        _scale(r1_v, j0 + 1)
        pltpu.sync_copy(r1_v, acc_sh.at[ds_v.at[j0 + 1]], add=True)
        return 0

    lax.fori_loop(0, CE // 2, _pair, 0)
    plsc.subcore_barrier()

    for k in range(5):
        pltpu.sync_copy(acc_sh.at[pl.ds(s * 625 + k * KE, KE)], cp_v)
        pltpu.sync_copy(cp_v, out_hbm.at[c, pl.ds(s * 625 + k * KE, KE)])


def _wstack_body(comp_ref, bases_ref, w_ref):
    r = pl.program_id(0)
    acc = jnp.zeros((D, D), f32)
    for b in range(NB + 1):
        acc = acc + comp_ref[r, b] * bases_ref[b]
    w_ref[0] = acc


def _tc_wstack(comp_ext, bases_ext):
    return pl.pallas_call(
        _wstack_body,
        out_shape=jax.ShapeDtypeStruct((R + 1, D, D), f32),
        grid=(R + 1,),
        in_specs=[
            pl.BlockSpec(memory_space=pltpu.SMEM),
            pl.BlockSpec((NB + 1, D, D), lambda r: (0, 0, 0)),
        ],
        out_specs=pl.BlockSpec((1, D, D), lambda r: (r, 0, 0)),
    )(comp_ext, bases_ext)


def _ymm_body(h_ref, w_ref, y_ref):
    y_ref[0] = jnp.dot(h_ref[...], w_ref[0], preferred_element_type=f32)


def _tc_ymm(h, wstack):
    return pl.pallas_call(
        _ymm_body,
        out_shape=jax.ShapeDtypeStruct((R + 1, N, D), f32),
        grid=(NT, R + 1),
        in_specs=[
            pl.BlockSpec((BN, D), lambda i, r: (i, 0)),
            pl.BlockSpec((1, D, D), lambda i, r: (r, 0, 0)),
        ],
        out_specs=pl.BlockSpec((1, BN, D), lambda i, r: (r, i, 0)),
    )(h, wstack)


def _postA_body(p_ref, y_ref, bias_ref, z_ref, st_ref, acc_ref):
    i = pl.program_id(0)
    @pl.when(i == 0)
    def _():
        acc_ref[...] = jnp.zeros((8, D), f32)
    z = p_ref[0] + p_ref[1] + y_ref[0] + bias_ref[...]
    z = jnp.maximum(z, 0.0)
    z_ref[...] = z
    acc_ref[0:1, :] = acc_ref[0:1, :] + jnp.sum(z, axis=0, keepdims=True)
    acc_ref[1:2, :] = acc_ref[1:2, :] + jnp.sum(z * z, axis=0, keepdims=True)
    @pl.when(i == NT - 1)
    def _():
        st_ref[...] = acc_ref[...]


def _tc_postA(p, y, bias2):
    return pl.pallas_call(
        _postA_body,
        out_shape=[jax.ShapeDtypeStruct((N, D), f32),
                   jax.ShapeDtypeStruct((8, D), f32)],
        grid=(NT,),
        in_specs=[
            pl.BlockSpec((NC, BN, D), lambda i: (0, i, 0)),
            pl.BlockSpec((1, BN, D), lambda i: (R, i, 0)),
            pl.BlockSpec((1, D), lambda i: (0, 0)),
        ],
        out_specs=[pl.BlockSpec((BN, D), lambda i: (i, 0)),
                   pl.BlockSpec((8, D), lambda i: (0, 0))],
        scratch_shapes=[pltpu.VMEM((8, D), f32)],
    )(p, y, bias2)


def _postB_body_res(z_ref, st_ref, g_ref, b_ref, res_ref, h_ref):
    mu = st_ref[0:1, :] / N
    var = st_ref[1:2, :] / N - mu * mu
    rstd = lax.rsqrt(var + 1e-5)
    h = (z_ref[...] - mu) * rstd * g_ref[...] + b_ref[...]
    h_ref[...] = h + res_ref[...]


def _postB_body(z_ref, st_ref, g_ref, b_ref, h_ref):
    mu = st_ref[0:1, :] / N
    var = st_ref[1:2, :] / N - mu * mu
    rstd = lax.rsqrt(var + 1e-5)
    h_ref[...] = (z_ref[...] - mu) * rstd * g_ref[...] + b_ref[...]


def _tc_postB(z, stats, g2, b2, res=None):
    vec = pl.BlockSpec((1, D), lambda i: (0, 0))
    blk = pl.BlockSpec((BN, D), lambda i: (i, 0))
    st = pl.BlockSpec((8, D), lambda i: (0, 0))
    if res is None:
        return pl.pallas_call(
            _postB_body,
            out_shape=jax.ShapeDtypeStruct((N, D), f32),
            grid=(NT,),
            in_specs=[blk, st, vec, vec],
            out_specs=blk,
        )(z, stats, g2, b2)
    return pl.pallas_call(
        _postB_body_res,
        out_shape=jax.ShapeDtypeStruct((N, D), f32),
        grid=(NT,),
        in_specs=[blk, st, vec, vec, blk],
        out_specs=blk,
    )(z, stats, g2, b2, res)


def _post3_body(p_ref, y_ref, bias_ref, o_ref):
    o_ref[...] = p_ref[0] + p_ref[1] + y_ref[0] + bias_ref[...]


def _tc_post3(p, y, bias2):
    return pl.pallas_call(
        _post3_body,
        out_shape=jax.ShapeDtypeStruct((N, D), f32),
        grid=(NT,),
        in_specs=[
            pl.BlockSpec((NC, BN, D), lambda i: (0, i, 0)),
            pl.BlockSpec((1, BN, D), lambda i: (R, i, 0)),
            pl.BlockSpec((1, D), lambda i: (0, 0)),
        ],
        out_specs=pl.BlockSpec((BN, D), lambda i: (i, 0)),
    )(p, y, bias2)


def _conv_layer(h, gidx3, dst3, w2, comp, bases, root):
    comp_ext = jnp.zeros((R + 1, NB + 1), f32)
    comp_ext = comp_ext.at[:R, :NB].set(comp).at[R, NB].set(1.0)
    bases_ext = jnp.concatenate([bases, root[None]], axis=0)
    wstack = _tc_wstack(comp_ext, bases_ext)
    y = _tc_ymm(h, wstack)
    p = _sc_edge(y.reshape((R + 1) * N, D), gidx3, dst3, w2)
    return p, y


def kernel(x, edge_index, edge_type,
           comp1, bases1, root1, bias1,
           comp2, bases2, root2, bias2,
           comp3, bases3, root3, bias3,
           g1, b1, g2, b2):
    src = edge_index[0]
    dst = edge_index[1]
    et = edge_type

    parts = _sc_count(dst, et)
    cinv = _tc_cinv(parts.reshape(NC, RN))
    gidx, w = _sc_prep(src, dst, et, cinv.reshape(RN))
    gidx3 = gidx.reshape(NW, CE, KE)
    dst3 = dst.reshape(NW, CE, KE)
    w2 = w.reshape(NW, EPT)

    p, y = _conv_layer(x, gidx3, dst3, w2, comp1, bases1, root1)
    z, st = _tc_postA(p, y, bias1[None, :])
    h1 = _tc_postB(z, st, g1[None, :], b1[None, :])
    p, y = _conv_layer(h1, gidx3, dst3, w2, comp2, bases2, root2)
    z, st = _tc_postA(p, y, bias2[None, :])
    h2 = _tc_postB(z, st, g2[None, :], b2[None, :], res=h1)
    p, y = _conv_layer(h2, gidx3, dst3, w2, comp3, bases3, root3)
    return _tc_post3(p, y, bias3[None, :])


# speedup vs baseline: 6.7035x; 6.7035x over previous
import functools

import jax
import jax.numpy as jnp
from jax import lax
from jax.experimental import pallas as pl
from jax.experimental.pallas import tpu as pltpu
from jax.experimental.pallas import tpu_sc as plsc

N = 10000
E = 320000
R = 8
NB = 8
D = 128
RN = R * N
NC = 2
NS = 16
NW = NC * NS
EPT = E // NW

KC = 80
CC = EPT // KC
KE = 80
CE = EPT // KE
NT = 10
BN = N // NT

f32 = jnp.float32
i32 = jnp.int32

_mesh = plsc.VectorSubcoreMesh(core_axis_name="c", subcore_axis_name="s")
_sc_params = pltpu.CompilerParams(use_tc_tiling_on_sc=False)


def _wid():
    return lax.axis_index("s") * NC + lax.axis_index("c")


RPT = RN // NS

@functools.partial(
    pl.kernel,
    out_type=jax.ShapeDtypeStruct((NC, RN, 16), f32),
    mesh=_mesh,
    compiler_params=_sc_params,
    scratch_types=[
        pltpu.VMEM((EPT,), i32),
        pltpu.VMEM((EPT,), i32),
        pltpu.VMEM((CC, KC), i32),
        pltpu.VMEM((KC, 16), f32),
        pltpu.VMEM_SHARED((RN, 16), f32),
        pltpu.VMEM((500, 16), f32),
        pltpu.SemaphoreType.DMA,
    ],
)
def _sc_count(dst_hbm, et_hbm, out_hbm, dst_v, et_v, seg_v, ones_v,
              cnt_sh, cp_v, sem):
    c = lax.axis_index("c")
    s = lax.axis_index("s")
    wid = _wid()
    base = wid * EPT
    pltpu.async_copy(dst_hbm.at[pl.ds(base, EPT)], dst_v, sem).wait()
    pltpu.async_copy(et_hbm.at[pl.ds(base, EPT)], et_v, sem).wait()

    zrow = jnp.zeros((16,), f32)
    onerow = jnp.ones((16,), f32)
    def _z(j, _):
        cp_v[j, :] = zrow
        return 0
    lax.fori_loop(0, 500, _z, 0)
    def _o(j, _):
        ones_v[j, :] = onerow
        return 0
    lax.fori_loop(0, KC, _o, 0)
    for k in range(RPT // 500):
        pltpu.sync_copy(cp_v, cnt_sh.at[pl.ds(s * RPT + k * 500, 500)])
    plsc.subcore_barrier()

    def _chunk(j, _):
        for k in range(KC // 16):
            off = j * KC + k * 16
            d16 = dst_v[pl.ds(off, 16)]
            t16 = et_v[pl.ds(off, 16)]
            seg_v[j, pl.ds(k * 16, 16)] = t16 * N + d16
        pltpu.sync_copy(ones_v, cnt_sh.at[seg_v.at[j]], add=True)
        return 0

    lax.fori_loop(0, CC, _chunk, 0)
    plsc.subcore_barrier()

    for k in range(RPT // 500):
        pltpu.sync_copy(cnt_sh.at[pl.ds(s * RPT + k * 500, 500)], cp_v)
        pltpu.sync_copy(cp_v, out_hbm.at[c, pl.ds(s * RPT + k * 500, 500)])


def _cinv_body(p_ref, o_ref):
    o_ref[...] = 1.0 / jnp.maximum(p_ref[0] + p_ref[1], 1.0)


def _tc_cinv(parts):
    p3 = parts.reshape(NC, RN // 128, 2048)
    return pl.pallas_call(
        _cinv_body,
        out_shape=jax.ShapeDtypeStruct((RN // 128, 2048), f32),
        grid=(8,),
        in_specs=[pl.BlockSpec((NC, RN // 128, 256), lambda i: (0, 0, i))],
        out_specs=pl.BlockSpec((RN // 128, 256), lambda i: (0, i)),
    )(p3)


@functools.partial(
    pl.kernel,
    out_type=[jax.ShapeDtypeStruct((NW, EPT), i32),
              jax.ShapeDtypeStruct((NW, EPT, 16), f32)],
    mesh=_mesh,
    compiler_params=_sc_params,
    scratch_types=[
        pltpu.VMEM((EPT,), i32),
        pltpu.VMEM((EPT,), i32),
        pltpu.VMEM((EPT,), i32),
        pltpu.VMEM((CC, KC), i32),
        pltpu.VMEM((KC, 16), f32),
        pltpu.SemaphoreType.DMA,
    ],
)
def _sc_prep(src_hbm, dst_hbm, et_hbm, winv_hbm, gidx_hbm, w_hbm,
             s_v, d_v, t_v, seg_v, wr_v, sem):
    wid = _wid()
    base = wid * EPT
    pltpu.async_copy(src_hbm.at[pl.ds(base, EPT)], s_v, sem).wait()
    pltpu.async_copy(dst_hbm.at[pl.ds(base, EPT)], d_v, sem).wait()
    pltpu.async_copy(et_hbm.at[pl.ds(base, EPT)], t_v, sem).wait()

    def _g(j, _):
        for k in range(KC // 16):
            off = j * KC + k * 16
            s16 = s_v[pl.ds(off, 16)]
            d16 = d_v[pl.ds(off, 16)]
            t16 = t_v[pl.ds(off, 16)]
            seg_v[j, pl.ds(k * 16, 16)] = t16 * N + d16
            s_v[pl.ds(off, 16)] = t16 * N + s16
        pltpu.async_copy(winv_hbm.at[seg_v.at[j]], wr_v, sem).wait()
        pltpu.sync_copy(wr_v, w_hbm.at[wid, pl.ds(j * KC, KC)])
        return 0

    lax.fori_loop(0, CC, _g, 0)
    pltpu.sync_copy(s_v, gidx_hbm.at[wid])


@functools.partial(
    pl.kernel,
    out_type=jax.ShapeDtypeStruct((NC, N, D), f32),
    mesh=_mesh,
    compiler_params=_sc_params,
    scratch_types=[
        pltpu.VMEM((CE, KE), i32),
        pltpu.VMEM((CE, KE), i32),
        pltpu.VMEM((KE, D), f32),
        pltpu.VMEM((KE, D), f32),
        pltpu.VMEM((KE, 16), f32),
        pltpu.VMEM((KE, 16), f32),
        pltpu.VMEM_SHARED((N, D), f32),
        pltpu.SemaphoreType.DMA,
        pltpu.SemaphoreType.DMA,
        pltpu.SemaphoreType.DMA,
        pltpu.SemaphoreType.DMA,
        pltpu.SemaphoreType.DMA,
    ],
)
def _sc_edge(y_hbm, gidx_hbm, dst_hbm, w_hbm, out_hbm,
             gi_v, ds_v, r0_v, r1_v, w0_v, w1_v, acc_sh,
             sem0, sem1, semw0, semw1, semc):
    c = lax.axis_index("c")
    s = lax.axis_index("s")
    wid = _wid()
    pltpu.async_copy(gidx_hbm.at[wid], gi_v, semc).wait()
    pltpu.async_copy(dst_hbm.at[wid], ds_v, semc).wait()

    zrow = jnp.zeros((16,), f32)
    def _z(j, _):
        for q in range(D // 16):
            r0_v[j, pl.ds(q * 16, 16)] = zrow
        return 0
    lax.fori_loop(0, KE, _z, 0)
    for k in range(7):
        pltpu.sync_copy(r0_v, acc_sh.at[pl.ds(s * 625 + k * KE, KE)])
    pltpu.sync_copy(r0_v.at[pl.ds(0, 65)],
                    acc_sh.at[pl.ds(s * 625 + 560, 65)])
    plsc.subcore_barrier()

    def _scale(slot_ref, w_ref):
        def _e(e, _):
            wv = w_ref[e, :]
            for q in range(D // 16):
                slot_ref[e, pl.ds(q * 16, 16)] = (
                    slot_ref[e, pl.ds(q * 16, 16)] * wv)
            return 0
        lax.fori_loop(0, KE, _e, 0)

    def _start(slot_ref, w_ref, semr, semw, j):
        pltpu.async_copy(y_hbm.at[gi_v.at[j]], slot_ref, semr)
        pltpu.async_copy(w_hbm.at[wid, pl.ds(j * KE, KE)], w_ref, semw)

    def _wait(slot_ref, w_ref, semr, semw, j):
        pltpu.make_async_copy(y_hbm.at[gi_v.at[j]], slot_ref, semr).wait()
        pltpu.make_async_copy(w_hbm.at[wid, pl.ds(j * KE, KE)],
                              w_ref, semw).wait()

    _start(r0_v, w0_v, sem0, semw0, 0)

    def _pair(jj, _):
        j0 = jj * 2
        _start(r1_v, w1_v, sem1, semw1, j0 + 1)
        _wait(r0_v, w0_v, sem0, semw0, j0)
        _scale(r0_v, w0_v)
        pltpu.sync_copy(r0_v, acc_sh.at[ds_v.at[j0]], add=True)
        _start(r0_v, w0_v, sem0, semw0, j0 + 2)
        _wait(r1_v, w1_v, sem1, semw1, j0 + 1)
        _scale(r1_v, w1_v)
        pltpu.sync_copy(r1_v, acc_sh.at[ds_v.at[j0 + 1]], add=True)
        return 0

    lax.fori_loop(0, (CE - 1) // 2, _pair, 0)
    _wait(r0_v, w0_v, sem0, semw0, CE - 1)
    _scale(r0_v, w0_v)
    pltpu.sync_copy(r0_v, acc_sh.at[ds_v.at[CE - 1]], add=True)
    plsc.subcore_barrier()

    for k in range(7):
        pltpu.sync_copy(acc_sh.at[pl.ds(s * 625 + k * KE, KE)], r0_v)
        pltpu.sync_copy(r0_v, out_hbm.at[c, pl.ds(s * 625 + k * KE, KE)])
    pltpu.sync_copy(acc_sh.at[pl.ds(s * 625 + 560, 65)],
                    r0_v.at[pl.ds(0, 65)])
    pltpu.sync_copy(r0_v.at[pl.ds(0, 65)],
                    out_hbm.at[c, pl.ds(s * 625 + 560, 65)])


def _wstack_body(comp_ref, bases_ref, w_ref):
    r = pl.program_id(0)
    acc = jnp.zeros((D, D), f32)
    for b in range(NB + 1):
        acc = acc + comp_ref[r, b] * bases_ref[b]
    w_ref[0] = acc


def _tc_wstack(comp_ext, bases_ext):
    return pl.pallas_call(
        _wstack_body,
        out_shape=jax.ShapeDtypeStruct((R + 1, D, D), f32),
        grid=(R + 1,),
        in_specs=[
            pl.BlockSpec(memory_space=pltpu.SMEM),
            pl.BlockSpec((NB + 1, D, D), lambda r: (0, 0, 0)),
        ],
        out_specs=pl.BlockSpec((1, D, D), lambda r: (r, 0, 0)),
    )(comp_ext, bases_ext)


def _ymm_body(h_ref, w_ref, y_ref):
    y_ref[0] = jnp.dot(h_ref[...], w_ref[0], preferred_element_type=f32)


def _tc_ymm(h, wstack):
    return pl.pallas_call(
        _ymm_body,
        out_shape=jax.ShapeDtypeStruct((R + 1, N, D), f32),
        grid=(NT, R + 1),
        in_specs=[
            pl.BlockSpec((BN, D), lambda i, r: (i, 0)),
            pl.BlockSpec((1, D, D), lambda i, r: (r, 0, 0)),
        ],
        out_specs=pl.BlockSpec((1, BN, D), lambda i, r: (r, i, 0)),
    )(h, wstack)


def _postA_body(p_ref, y_ref, bias_ref, z_ref, st_ref, acc_ref):
    i = pl.program_id(0)
    @pl.when(i == 0)
    def _():
        acc_ref[...] = jnp.zeros((8, D), f32)
    z = p_ref[0] + p_ref[1] + y_ref[0] + bias_ref[...]
    z = jnp.maximum(z, 0.0)
    z_ref[...] = z
    acc_ref[0:1, :] = acc_ref[0:1, :] + jnp.sum(z, axis=0, keepdims=True)
    acc_ref[1:2, :] = acc_ref[1:2, :] + jnp.sum(z * z, axis=0, keepdims=True)
    @pl.when(i == NT - 1)
    def _():
        st_ref[...] = acc_ref[...]


def _tc_postA(p, y, bias2):
    return pl.pallas_call(
        _postA_body,
        out_shape=[jax.ShapeDtypeStruct((N, D), f32),
                   jax.ShapeDtypeStruct((8, D), f32)],
        grid=(NT,),
        in_specs=[
            pl.BlockSpec((NC, BN, D), lambda i: (0, i, 0)),
            pl.BlockSpec((1, BN, D), lambda i: (R, i, 0)),
            pl.BlockSpec((1, D), lambda i: (0, 0)),
        ],
        out_specs=[pl.BlockSpec((BN, D), lambda i: (i, 0)),
                   pl.BlockSpec((8, D), lambda i: (0, 0))],
        scratch_shapes=[pltpu.VMEM((8, D), f32)],
    )(p, y, bias2)


def _postB_body_res(z_ref, st_ref, g_ref, b_ref, res_ref, h_ref):
    mu = st_ref[0:1, :] / N
    var = st_ref[1:2, :] / N - mu * mu
    rstd = lax.rsqrt(var + 1e-5)
    h = (z_ref[...] - mu) * rstd * g_ref[...] + b_ref[...]
    h_ref[...] = h + res_ref[...]


def _postB_body(z_ref, st_ref, g_ref, b_ref, h_ref):
    mu = st_ref[0:1, :] / N
    var = st_ref[1:2, :] / N - mu * mu
    rstd = lax.rsqrt(var + 1e-5)
    h_ref[...] = (z_ref[...] - mu) * rstd * g_ref[...] + b_ref[...]


def _tc_postB(z, stats, g2, b2, res=None):
    vec = pl.BlockSpec((1, D), lambda i: (0, 0))
    blk = pl.BlockSpec((BN, D), lambda i: (i, 0))
    st = pl.BlockSpec((8, D), lambda i: (0, 0))
    if res is None:
        return pl.pallas_call(
            _postB_body,
            out_shape=jax.ShapeDtypeStruct((N, D), f32),
            grid=(NT,),
            in_specs=[blk, st, vec, vec],
            out_specs=blk,
        )(z, stats, g2, b2)
    return pl.pallas_call(
        _postB_body_res,
        out_shape=jax.ShapeDtypeStruct((N, D), f32),
        grid=(NT,),
        in_specs=[blk, st, vec, vec, blk],
        out_specs=blk,
    )(z, stats, g2, b2, res)


def _post3_body(p_ref, y_ref, bias_ref, o_ref):
    o_ref[...] = p_ref[0] + p_ref[1] + y_ref[0] + bias_ref[...]


def _tc_post3(p, y, bias2):
    return pl.pallas_call(
        _post3_body,
        out_shape=jax.ShapeDtypeStruct((N, D), f32),
        grid=(NT,),
        in_specs=[
            pl.BlockSpec((NC, BN, D), lambda i: (0, i, 0)),
            pl.BlockSpec((1, BN, D), lambda i: (R, i, 0)),
            pl.BlockSpec((1, D), lambda i: (0, 0)),
        ],
        out_specs=pl.BlockSpec((BN, D), lambda i: (i, 0)),
    )(p, y, bias2)


def _conv_layer(h, gidx3, dst3, w2, comp, bases, root):
    comp_ext = jnp.zeros((R + 1, NB + 1), f32)
    comp_ext = comp_ext.at[:R, :NB].set(comp).at[R, NB].set(1.0)
    bases_ext = jnp.concatenate([bases, root[None]], axis=0)
    wstack = _tc_wstack(comp_ext, bases_ext)
    y = _tc_ymm(h, wstack)
    p = _sc_edge(y.reshape((R + 1) * N, D), gidx3, dst3, w2)
    return p, y


def kernel(x, edge_index, edge_type,
           comp1, bases1, root1, bias1,
           comp2, bases2, root2, bias2,
           comp3, bases3, root3, bias3,
           g1, b1, g2, b2):
    src = edge_index[0]
    dst = edge_index[1]
    et = edge_type

    parts = _sc_count(dst, et)
    winv = _tc_cinv(parts)
    gidx, w_rep = _sc_prep(src, dst, et, winv.reshape(RN, 16))
    gidx3 = gidx.reshape(NW, CE, KE)
    dst3 = dst.reshape(NW, CE, KE)
    w2 = w_rep

    p, y = _conv_layer(x, gidx3, dst3, w2, comp1, bases1, root1)
    z, st = _tc_postA(p, y, bias1[None, :])
    h1 = _tc_postB(z, st, g1[None, :], b1[None, :])
    p, y = _conv_layer(h1, gidx3, dst3, w2, comp2, bases2, root2)
    z, st = _tc_postA(p, y, bias2[None, :])
    h2 = _tc_postB(z, st, g2[None, :], b2[None, :], res=h1)
    p, y = _conv_layer(h2, gidx3, dst3, w2, comp3, bases3, root3)
    return _tc_post3(p, y, bias3[None, :])


# speedup vs baseline: 6.7783x; 1.0112x over previous
import functools

import jax
import jax.numpy as jnp
from jax import lax
from jax.experimental import pallas as pl
from jax.experimental.pallas import tpu as pltpu
from jax.experimental.pallas import tpu_sc as plsc

N = 10000
E = 320000
R = 8
NB = 8
D = 128
RN = R * N
NC = 2
NS = 16
NW = NC * NS
EPT = E // NW

KC = 80
CC = EPT // KC
KE = 80
CE = EPT // KE
NT = 10
BN = N // NT

f32 = jnp.float32
i32 = jnp.int32

_mesh = plsc.VectorSubcoreMesh(core_axis_name="c", subcore_axis_name="s")
_sc_params = pltpu.CompilerParams(use_tc_tiling_on_sc=False)


def _wid():
    return lax.axis_index("s") * NC + lax.axis_index("c")


RPT = RN // NS

@functools.partial(
    pl.kernel,
    out_type=jax.ShapeDtypeStruct((NC, RN, 16), f32),
    mesh=_mesh,
    compiler_params=_sc_params,
    scratch_types=[
        pltpu.VMEM((EPT,), i32),
        pltpu.VMEM((EPT,), i32),
        pltpu.VMEM((CC, KC), i32),
        pltpu.VMEM((KC, 16), f32),
        pltpu.VMEM_SHARED((RN, 16), f32),
        pltpu.VMEM((500, 16), f32),
        pltpu.SemaphoreType.DMA,
    ],
)
def _sc_count(dst_hbm, et_hbm, out_hbm, dst_v, et_v, seg_v, ones_v,
              cnt_sh, cp_v, sem):
    c = lax.axis_index("c")
    s = lax.axis_index("s")
    wid = _wid()
    base = wid * EPT
    pltpu.async_copy(dst_hbm.at[pl.ds(base, EPT)], dst_v, sem).wait()
    pltpu.async_copy(et_hbm.at[pl.ds(base, EPT)], et_v, sem).wait()

    zrow = jnp.zeros((16,), f32)
    onerow = jnp.ones((16,), f32)
    def _z(j, _):
        cp_v[j, :] = zrow
        return 0
    lax.fori_loop(0, 500, _z, 0)
    def _o(j, _):
        ones_v[j, :] = onerow
        return 0
    lax.fori_loop(0, KC, _o, 0)
    for k in range(RPT // 500):
        pltpu.sync_copy(cp_v, cnt_sh.at[pl.ds(s * RPT + k * 500, 500)])
    plsc.subcore_barrier()

    def _chunk(j, _):
        for k in range(KC // 16):
            off = j * KC + k * 16
            d16 = dst_v[pl.ds(off, 16)]
            t16 = et_v[pl.ds(off, 16)]
            seg_v[j, pl.ds(k * 16, 16)] = t16 * N + d16
        pltpu.sync_copy(ones_v, cnt_sh.at[seg_v.at[j]], add=True)
        return 0

    lax.fori_loop(0, CC, _chunk, 0)
    plsc.subcore_barrier()

    for k in range(RPT // 500):
        pltpu.sync_copy(cnt_sh.at[pl.ds(s * RPT + k * 500, 500)], cp_v)
        pltpu.sync_copy(cp_v, out_hbm.at[c, pl.ds(s * RPT + k * 500, 500)])


def _cinv_body(p_ref, o_ref):
    o_ref[...] = 1.0 / jnp.maximum(p_ref[0] + p_ref[1], 1.0)


def _tc_cinv(parts):
    p3 = parts.reshape(NC, RN // 128, 2048)
    return pl.pallas_call(
        _cinv_body,
        out_shape=jax.ShapeDtypeStruct((RN // 128, 2048), f32),
        grid=(8,),
        in_specs=[pl.BlockSpec((NC, RN // 128, 256), lambda i: (0, 0, i))],
        out_specs=pl.BlockSpec((RN // 128, 256), lambda i: (0, i)),
    )(p3)


@functools.partial(
    pl.kernel,
    out_type=[jax.ShapeDtypeStruct((NW, EPT), i32),
              jax.ShapeDtypeStruct((NW, EPT, 16), f32)],
    mesh=_mesh,
    compiler_params=_sc_params,
    scratch_types=[
        pltpu.VMEM((EPT,), i32),
        pltpu.VMEM((EPT,), i32),
        pltpu.VMEM((EPT,), i32),
        pltpu.VMEM((CC, KC), i32),
        pltpu.VMEM((KC, 16), f32),
        pltpu.SemaphoreType.DMA,
    ],
)
def _sc_prep(src_hbm, dst_hbm, et_hbm, winv_hbm, gidx_hbm, w_hbm,
             s_v, d_v, t_v, seg_v, wr_v, sem):
    wid = _wid()
    base = wid * EPT
    pltpu.async_copy(src_hbm.at[pl.ds(base, EPT)], s_v, sem).wait()
    pltpu.async_copy(dst_hbm.at[pl.ds(base, EPT)], d_v, sem).wait()
    pltpu.async_copy(et_hbm.at[pl.ds(base, EPT)], t_v, sem).wait()

    def _g(j, _):
        for k in range(KC // 16):
            off = j * KC + k * 16
            s16 = s_v[pl.ds(off, 16)]
            d16 = d_v[pl.ds(off, 16)]
            t16 = t_v[pl.ds(off, 16)]
            seg_v[j, pl.ds(k * 16, 16)] = t16 * N + d16
            s_v[pl.ds(off, 16)] = t16 * N + s16
        pltpu.async_copy(winv_hbm.at[seg_v.at[j]], wr_v, sem).wait()
        pltpu.sync_copy(wr_v, w_hbm.at[wid, pl.ds(j * KC, KC)])
        return 0

    lax.fori_loop(0, CC, _g, 0)
    pltpu.sync_copy(s_v, gidx_hbm.at[wid])


@functools.partial(
    pl.kernel,
    out_type=jax.ShapeDtypeStruct((NC, N, D), f32),
    mesh=_mesh,
    compiler_params=_sc_params,
    scratch_types=[
        pltpu.VMEM((CE, KE), i32),
        pltpu.VMEM((CE, KE), i32),
        pltpu.VMEM((KE, D), f32),
        pltpu.VMEM((KE, D), f32),
        pltpu.VMEM((KE, 16), f32),
        pltpu.VMEM((KE, 16), f32),
        pltpu.VMEM_SHARED((N, D), f32),
        pltpu.SemaphoreType.DMA,
        pltpu.SemaphoreType.DMA,
        pltpu.SemaphoreType.DMA,
        pltpu.SemaphoreType.DMA,
        pltpu.SemaphoreType.DMA,
        pltpu.SemaphoreType.DMA,
        pltpu.SemaphoreType.DMA,
    ],
)
def _sc_edge(y_hbm, gidx_hbm, dst_hbm, w_hbm, out_hbm,
             gi_v, ds_v, r0_v, r1_v, w0_v, w1_v, acc_sh,
             sem0, sem1, semw0, semw1, sems0, sems1, semc):
    c = lax.axis_index("c")
    s = lax.axis_index("s")
    wid = _wid()
    pltpu.async_copy(gidx_hbm.at[wid], gi_v, semc).wait()
    pltpu.async_copy(dst_hbm.at[wid], ds_v, semc).wait()

    zrow = jnp.zeros((16,), f32)
    def _z(j, _):
        for q in range(D // 16):
            r0_v[j, pl.ds(q * 16, 16)] = zrow
        return 0
    lax.fori_loop(0, KE, _z, 0)
    for k in range(7):
        pltpu.sync_copy(r0_v, acc_sh.at[pl.ds(s * 625 + k * KE, KE)])
    pltpu.sync_copy(r0_v.at[pl.ds(0, 65)],
                    acc_sh.at[pl.ds(s * 625 + 560, 65)])
    plsc.subcore_barrier()

    def _scale(slot_ref, w_ref):
        @plsc.parallel_loop(0, KE, unroll=4)
        def _e(e):
            wv = w_ref[e, :]
            for q in range(D // 16):
                slot_ref[e, pl.ds(q * 16, 16)] = (
                    slot_ref[e, pl.ds(q * 16, 16)] * wv)

    def _start(slot_ref, w_ref, semr, semw, j):
        pltpu.async_copy(y_hbm.at[gi_v.at[j]], slot_ref, semr)
        pltpu.async_copy(w_hbm.at[wid, pl.ds(j * KE, KE)], w_ref, semw)

    def _wait(slot_ref, w_ref, semr, semw, j):
        pltpu.make_async_copy(y_hbm.at[gi_v.at[j]], slot_ref, semr).wait()
        pltpu.make_async_copy(w_hbm.at[wid, pl.ds(j * KE, KE)],
                              w_ref, semw).wait()

    _start(r0_v, w0_v, sem0, semw0, 0)
    _start(r1_v, w1_v, sem1, semw1, 1)

    def _pair(jj, _):
        j0 = jj * 2
        _wait(r0_v, w0_v, sem0, semw0, j0)
        _scale(r0_v, w0_v)
        pltpu.async_copy(r0_v, acc_sh.at[ds_v.at[j0]], sems0, add=True)
        _wait(r1_v, w1_v, sem1, semw1, j0 + 1)
        _scale(r1_v, w1_v)
        pltpu.async_copy(r1_v, acc_sh.at[ds_v.at[j0 + 1]], sems1, add=True)
        pltpu.make_async_copy(r0_v, acc_sh.at[ds_v.at[j0]], sems0).wait()
        _start(r0_v, w0_v, sem0, semw0, j0 + 2)
        pltpu.make_async_copy(r1_v, acc_sh.at[ds_v.at[j0 + 1]], sems1).wait()
        @pl.when(j0 + 3 < CE)
        def _():
            _start(r1_v, w1_v, sem1, semw1, j0 + 3)
        return 0

    lax.fori_loop(0, (CE - 1) // 2, _pair, 0)
    _wait(r0_v, w0_v, sem0, semw0, CE - 1)
    _scale(r0_v, w0_v)
    pltpu.sync_copy(r0_v, acc_sh.at[ds_v.at[CE - 1]], add=True)
    plsc.subcore_barrier()

    for k in range(7):
        pltpu.sync_copy(acc_sh.at[pl.ds(s * 625 + k * KE, KE)], r0_v)
        pltpu.sync_copy(r0_v, out_hbm.at[c, pl.ds(s * 625 + k * KE, KE)])
    pltpu.sync_copy(acc_sh.at[pl.ds(s * 625 + 560, 65)],
                    r0_v.at[pl.ds(0, 65)])
    pltpu.sync_copy(r0_v.at[pl.ds(0, 65)],
                    out_hbm.at[c, pl.ds(s * 625 + 560, 65)])


def _wstack_body(comp_ref, bases_ref, w_ref):
    r = pl.program_id(0)
    acc = jnp.zeros((D, D), f32)
    for b in range(NB + 1):
        acc = acc + comp_ref[r, b] * bases_ref[b]
    w_ref[0] = acc


def _tc_wstack(comp_ext, bases_ext):
    return pl.pallas_call(
        _wstack_body,
        out_shape=jax.ShapeDtypeStruct((R + 1, D, D), f32),
        grid=(R + 1,),
        in_specs=[
            pl.BlockSpec(memory_space=pltpu.SMEM),
            pl.BlockSpec((NB + 1, D, D), lambda r: (0, 0, 0)),
        ],
        out_specs=pl.BlockSpec((1, D, D), lambda r: (r, 0, 0)),
    )(comp_ext, bases_ext)


def _ymm_body(h_ref, w_ref, y_ref):
    y_ref[0] = jnp.dot(h_ref[...], w_ref[0], preferred_element_type=f32)


def _tc_ymm(h, wstack):
    return pl.pallas_call(
        _ymm_body,
        out_shape=jax.ShapeDtypeStruct((R + 1, N, D), f32),
        grid=(NT, R + 1),
        in_specs=[
            pl.BlockSpec((BN, D), lambda i, r: (i, 0)),
            pl.BlockSpec((1, D, D), lambda i, r: (r, 0, 0)),
        ],
        out_specs=pl.BlockSpec((1, BN, D), lambda i, r: (r, i, 0)),
    )(h, wstack)


def _postA_body(p_ref, y_ref, bias_ref, z_ref, st_ref, acc_ref):
    i = pl.program_id(0)
    @pl.when(i == 0)
    def _():
        acc_ref[...] = jnp.zeros((8, D), f32)
    z = p_ref[0] + p_ref[1] + y_ref[0] + bias_ref[...]
    z = jnp.maximum(z, 0.0)
    z_ref[...] = z
    acc_ref[0:1, :] = acc_ref[0:1, :] + jnp.sum(z, axis=0, keepdims=True)
    acc_ref[1:2, :] = acc_ref[1:2, :] + jnp.sum(z * z, axis=0, keepdims=True)
    @pl.when(i == NT - 1)
    def _():
        st_ref[...] = acc_ref[...]


def _tc_postA(p, y, bias2):
    return pl.pallas_call(
        _postA_body,
        out_shape=[jax.ShapeDtypeStruct((N, D), f32),
                   jax.ShapeDtypeStruct((8, D), f32)],
        grid=(NT,),
        in_specs=[
            pl.BlockSpec((NC, BN, D), lambda i: (0, i, 0)),
            pl.BlockSpec((1, BN, D), lambda i: (R, i, 0)),
            pl.BlockSpec((1, D), lambda i: (0, 0)),
        ],
        out_specs=[pl.BlockSpec((BN, D), lambda i: (i, 0)),
                   pl.BlockSpec((8, D), lambda i: (0, 0))],
        scratch_shapes=[pltpu.VMEM((8, D), f32)],
    )(p, y, bias2)


def _postB_body_res(z_ref, st_ref, g_ref, b_ref, res_ref, h_ref):
    mu = st_ref[0:1, :] / N
    var = st_ref[1:2, :] / N - mu * mu
    rstd = lax.rsqrt(var + 1e-5)
    h = (z_ref[...] - mu) * rstd * g_ref[...] + b_ref[...]
    h_ref[...] = h + res_ref[...]


def _postB_body(z_ref, st_ref, g_ref, b_ref, h_ref):
    mu = st_ref[0:1, :] / N
    var = st_ref[1:2, :] / N - mu * mu
    rstd = lax.rsqrt(var + 1e-5)
    h_ref[...] = (z_ref[...] - mu) * rstd * g_ref[...] + b_ref[...]


def _tc_postB(z, stats, g2, b2, res=None):
    vec = pl.BlockSpec((1, D), lambda i: (0, 0))
    blk = pl.BlockSpec((BN, D), lambda i: (i, 0))
    st = pl.BlockSpec((8, D), lambda i: (0, 0))
    if res is None:
        return pl.pallas_call(
            _postB_body,
            out_shape=jax.ShapeDtypeStruct((N, D), f32),
            grid=(NT,),
            in_specs=[blk, st, vec, vec],
            out_specs=blk,
        )(z, stats, g2, b2)
    return pl.pallas_call(
        _postB_body_res,
        out_shape=jax.ShapeDtypeStruct((N, D), f32),
        grid=(NT,),
        in_specs=[blk, st, vec, vec, blk],
        out_specs=blk,
    )(z, stats, g2, b2, res)


def _post3_body(p_ref, y_ref, bias_ref, o_ref):
    o_ref[...] = p_ref[0] + p_ref[1] + y_ref[0] + bias_ref[...]


def _tc_post3(p, y, bias2):
    return pl.pallas_call(
        _post3_body,
        out_shape=jax.ShapeDtypeStruct((N, D), f32),
        grid=(NT,),
        in_specs=[
            pl.BlockSpec((NC, BN, D), lambda i: (0, i, 0)),
            pl.BlockSpec((1, BN, D), lambda i: (R, i, 0)),
            pl.BlockSpec((1, D), lambda i: (0, 0)),
        ],
        out_specs=pl.BlockSpec((BN, D), lambda i: (i, 0)),
    )(p, y, bias2)


def _conv_layer(h, gidx3, dst3, w2, comp, bases, root):
    comp_ext = jnp.zeros((R + 1, NB + 1), f32)
    comp_ext = comp_ext.at[:R, :NB].set(comp).at[R, NB].set(1.0)
    bases_ext = jnp.concatenate([bases, root[None]], axis=0)
    wstack = _tc_wstack(comp_ext, bases_ext)
    y = _tc_ymm(h, wstack)
    p = _sc_edge(y.reshape((R + 1) * N, D), gidx3, dst3, w2)
    return p, y


def kernel(x, edge_index, edge_type,
           comp1, bases1, root1, bias1,
           comp2, bases2, root2, bias2,
           comp3, bases3, root3, bias3,
           g1, b1, g2, b2):
    src = edge_index[0]
    dst = edge_index[1]
    et = edge_type

    parts = _sc_count(dst, et)
    winv = _tc_cinv(parts)
    gidx, w_rep = _sc_prep(src, dst, et, winv.reshape(RN, 16))
    gidx3 = gidx.reshape(NW, CE, KE)
    dst3 = dst.reshape(NW, CE, KE)
    w2 = w_rep

    p, y = _conv_layer(x, gidx3, dst3, w2, comp1, bases1, root1)
    z, st = _tc_postA(p, y, bias1[None, :])
    h1 = _tc_postB(z, st, g1[None, :], b1[None, :])
    p, y = _conv_layer(h1, gidx3, dst3, w2, comp2, bases2, root2)
    z, st = _tc_postA(p, y, bias2[None, :])
    h2 = _tc_postB(z, st, g2[None, :], b2[None, :], res=h1)
    p, y = _conv_layer(h2, gidx3, dst3, w2, comp3, bases3, root3)
    return _tc_post3(p, y, bias3[None, :])


# speedup vs baseline: 7.1039x; 1.0480x over previous
import functools

import jax
import jax.numpy as jnp
from jax import lax
from jax.experimental import pallas as pl
from jax.experimental.pallas import tpu as pltpu
from jax.experimental.pallas import tpu_sc as plsc

N = 10000
E = 320000
R = 8
NB = 8
D = 128
RN = R * N
NC = 2
NS = 16
NW = NC * NS
EPT = E // NW

KC = 80
CC = EPT // KC
KE = 80
CE = EPT // KE
NT = 10
BN = N // NT

f32 = jnp.float32
i32 = jnp.int32

_mesh = plsc.VectorSubcoreMesh(core_axis_name="c", subcore_axis_name="s")
_sc_params = pltpu.CompilerParams(use_tc_tiling_on_sc=False)


def _wid():
    return lax.axis_index("s") * NC + lax.axis_index("c")


RPT = RN // NS

@functools.partial(
    pl.kernel,
    out_type=jax.ShapeDtypeStruct((NC, RN, 16), f32),
    mesh=_mesh,
    compiler_params=_sc_params,
    scratch_types=[
        pltpu.VMEM((EPT,), i32),
        pltpu.VMEM((EPT,), i32),
        pltpu.VMEM((CC, KC), i32),
        pltpu.VMEM((KC, 16), f32),
        pltpu.VMEM_SHARED((RN, 16), f32),
        pltpu.VMEM((500, 16), f32),
        pltpu.SemaphoreType.DMA,
    ],
)
def _sc_count(dst_hbm, et_hbm, out_hbm, dst_v, et_v, seg_v, ones_v,
              cnt_sh, cp_v, sem):
    c = lax.axis_index("c")
    s = lax.axis_index("s")
    wid = _wid()
    base = wid * EPT
    pltpu.async_copy(dst_hbm.at[pl.ds(base, EPT)], dst_v, sem).wait()
    pltpu.async_copy(et_hbm.at[pl.ds(base, EPT)], et_v, sem).wait()

    zrow = jnp.zeros((16,), f32)
    onerow = jnp.ones((16,), f32)
    def _z(j, _):
        cp_v[j, :] = zrow
        return 0
    lax.fori_loop(0, 500, _z, 0)
    def _o(j, _):
        ones_v[j, :] = onerow
        return 0
    lax.fori_loop(0, KC, _o, 0)
    for k in range(RPT // 500):
        pltpu.sync_copy(cp_v, cnt_sh.at[pl.ds(s * RPT + k * 500, 500)])
    plsc.subcore_barrier()

    def _chunk(j, _):
        for k in range(KC // 16):
            off = j * KC + k * 16
            d16 = dst_v[pl.ds(off, 16)]
            t16 = et_v[pl.ds(off, 16)]
            seg_v[j, pl.ds(k * 16, 16)] = t16 * N + d16
        pltpu.sync_copy(ones_v, cnt_sh.at[seg_v.at[j]], add=True)
        return 0

    lax.fori_loop(0, CC, _chunk, 0)
    plsc.subcore_barrier()

    for k in range(RPT // 500):
        pltpu.sync_copy(cnt_sh.at[pl.ds(s * RPT + k * 500, 500)], cp_v)
        pltpu.sync_copy(cp_v, out_hbm.at[c, pl.ds(s * RPT + k * 500, 500)])


def _cinv_body(p_ref, o_ref):
    o_ref[...] = 1.0 / jnp.maximum(p_ref[0] + p_ref[1], 1.0)


def _tc_cinv(parts):
    p3 = parts.reshape(NC, RN // 128, 2048)
    return pl.pallas_call(
        _cinv_body,
        out_shape=jax.ShapeDtypeStruct((RN // 128, 2048), f32),
        grid=(8,),
        in_specs=[pl.BlockSpec((NC, RN // 128, 256), lambda i: (0, 0, i))],
        out_specs=pl.BlockSpec((RN // 128, 256), lambda i: (0, i)),
    )(p3)


@functools.partial(
    pl.kernel,
    out_type=[jax.ShapeDtypeStruct((NW, EPT), i32),
              jax.ShapeDtypeStruct((NW, EPT, 16), f32)],
    mesh=_mesh,
    compiler_params=_sc_params,
    scratch_types=[
        pltpu.VMEM((EPT,), i32),
        pltpu.VMEM((EPT,), i32),
        pltpu.VMEM((EPT,), i32),
        pltpu.VMEM((CC, KC), i32),
        pltpu.VMEM((KC, 16), f32),
        pltpu.VMEM((KC, 16), f32),
        pltpu.VMEM((KC, 16), f32),
        pltpu.VMEM((KC, 16), f32),
        pltpu.SemaphoreType.DMA,
        pltpu.SemaphoreType.DMA,
        pltpu.SemaphoreType.DMA,
        pltpu.SemaphoreType.DMA,
        pltpu.SemaphoreType.DMA,
        pltpu.SemaphoreType.DMA,
        pltpu.SemaphoreType.DMA,
        pltpu.SemaphoreType.DMA,
        pltpu.SemaphoreType.DMA,
    ],
)
def _sc_prep(src_hbm, dst_hbm, et_hbm, winv_hbm, gidx_hbm, w_hbm,
             s_v, d_v, t_v, seg_v, w0, w1, w2, w3,
             sg0, sg1, sg2, sg3, ss0, ss1, ss2, ss3, semc):
    wid = _wid()
    base = wid * EPT
    pltpu.async_copy(src_hbm.at[pl.ds(base, EPT)], s_v, semc).wait()
    pltpu.async_copy(dst_hbm.at[pl.ds(base, EPT)], d_v, semc).wait()
    pltpu.async_copy(et_hbm.at[pl.ds(base, EPT)], t_v, semc).wait()

    @plsc.parallel_loop(0, CC * (KC // 16), unroll=4)
    def _c(g):
        off = g * 16
        s16 = s_v[pl.ds(off, 16)]
        d16 = d_v[pl.ds(off, 16)]
        t16 = t_v[pl.ds(off, 16)]
        seg_v[g // (KC // 16), pl.ds((g % (KC // 16)) * 16, 16)] = (
            t16 * N + d16)
        s_v[pl.ds(off, 16)] = t16 * N + s16

    slots = (w0, w1, w2, w3)
    gsems = (sg0, sg1, sg2, sg3)
    ssems = (ss0, ss1, ss2, ss3)

    def _g(j, q):
        pltpu.async_copy(winv_hbm.at[seg_v.at[j]], slots[q], gsems[q])

    def _gw(j, q):
        pltpu.make_async_copy(winv_hbm.at[seg_v.at[j]], slots[q],
                              gsems[q]).wait()

    def _s(j, q):
        pltpu.async_copy(slots[q], w_hbm.at[wid, pl.ds(j * KC, KC)],
                         ssems[q])

    def _sw(j, q):
        pltpu.make_async_copy(slots[q], w_hbm.at[wid, pl.ds(j * KC, KC)],
                              ssems[q]).wait()

    for q in range(4):
        _g(q, q)

    def _quad(jj, _):
        j0 = jj * 4
        for q in range(4):
            _gw(j0 + q, q)
            _s(j0 + q, q)
        for q in range(4):
            _sw(j0 + q, q)
            @pl.when(j0 + 4 + q < CC)
            def _():
                _g(j0 + 4 + q, q)
        return 0

    lax.fori_loop(0, CC // 4, _quad, 0)
    _gw(CC - 1, 0)
    _s(CC - 1, 0)
    _sw(CC - 1, 0)
    pltpu.sync_copy(s_v, gidx_hbm.at[wid])


@functools.partial(
    pl.kernel,
    out_type=jax.ShapeDtypeStruct((NC, N, D), f32),
    mesh=_mesh,
    compiler_params=_sc_params,
    scratch_types=[
        pltpu.VMEM((CE, KE), i32),
        pltpu.VMEM((CE, KE), i32),
        pltpu.VMEM((KE, D), f32),
        pltpu.VMEM((KE, D), f32),
        pltpu.VMEM((KE, 16), f32),
        pltpu.VMEM((KE, 16), f32),
        pltpu.VMEM_SHARED((N, D), f32),
        pltpu.SemaphoreType.DMA,
        pltpu.SemaphoreType.DMA,
        pltpu.SemaphoreType.DMA,
        pltpu.SemaphoreType.DMA,
        pltpu.SemaphoreType.DMA,
        pltpu.SemaphoreType.DMA,
        pltpu.SemaphoreType.DMA,
    ],
)
def _sc_edge(y_hbm, gidx_hbm, dst_hbm, w_hbm, out_hbm,
             gi_v, ds_v, r0_v, r1_v, w0_v, w1_v, acc_sh,
             sem0, sem1, semw0, semw1, sems0, sems1, semc):
    c = lax.axis_index("c")
    s = lax.axis_index("s")
    wid = _wid()
    pltpu.async_copy(gidx_hbm.at[wid], gi_v, semc).wait()
    pltpu.async_copy(dst_hbm.at[wid], ds_v, semc).wait()

    zrow = jnp.zeros((16,), f32)
    def _z(j, _):
        for q in range(D // 16):
            r0_v[j, pl.ds(q * 16, 16)] = zrow
        return 0
    lax.fori_loop(0, KE, _z, 0)
    for k in range(7):
        pltpu.sync_copy(r0_v, acc_sh.at[pl.ds(s * 625 + k * KE, KE)])
    pltpu.sync_copy(r0_v.at[pl.ds(0, 65)],
                    acc_sh.at[pl.ds(s * 625 + 560, 65)])
    plsc.subcore_barrier()

    def _scale(slot_ref, w_ref):
        @plsc.parallel_loop(0, KE, unroll=4)
        def _e(e):
            wv = w_ref[e, :]
            for q in range(D // 16):
                slot_ref[e, pl.ds(q * 16, 16)] = (
                    slot_ref[e, pl.ds(q * 16, 16)] * wv)

    def _start(slot_ref, w_ref, semr, semw, j):
        pltpu.async_copy(y_hbm.at[gi_v.at[j]], slot_ref, semr)
        pltpu.async_copy(w_hbm.at[wid, pl.ds(j * KE, KE)], w_ref, semw)

    def _wait(slot_ref, w_ref, semr, semw, j):
        pltpu.make_async_copy(y_hbm.at[gi_v.at[j]], slot_ref, semr).wait()
        pltpu.make_async_copy(w_hbm.at[wid, pl.ds(j * KE, KE)],
                              w_ref, semw).wait()

    _start(r0_v, w0_v, sem0, semw0, 0)
    _start(r1_v, w1_v, sem1, semw1, 1)

    def _pair(jj, _):
        j0 = jj * 2
        _wait(r0_v, w0_v, sem0, semw0, j0)
        _scale(r0_v, w0_v)
        pltpu.async_copy(r0_v, acc_sh.at[ds_v.at[j0]], sems0, add=True)
        _wait(r1_v, w1_v, sem1, semw1, j0 + 1)
        _scale(r1_v, w1_v)
        pltpu.async_copy(r1_v, acc_sh.at[ds_v.at[j0 + 1]], sems1, add=True)
        pltpu.make_async_copy(r0_v, acc_sh.at[ds_v.at[j0]], sems0).wait()
        _start(r0_v, w0_v, sem0, semw0, j0 + 2)
        pltpu.make_async_copy(r1_v, acc_sh.at[ds_v.at[j0 + 1]], sems1).wait()
        @pl.when(j0 + 3 < CE)
        def _():
            _start(r1_v, w1_v, sem1, semw1, j0 + 3)
        return 0

    lax.fori_loop(0, (CE - 1) // 2, _pair, 0)
    _wait(r0_v, w0_v, sem0, semw0, CE - 1)
    _scale(r0_v, w0_v)
    pltpu.sync_copy(r0_v, acc_sh.at[ds_v.at[CE - 1]], add=True)
    plsc.subcore_barrier()

    for k in range(7):
        pltpu.sync_copy(acc_sh.at[pl.ds(s * 625 + k * KE, KE)], r0_v)
        pltpu.sync_copy(r0_v, out_hbm.at[c, pl.ds(s * 625 + k * KE, KE)])
    pltpu.sync_copy(acc_sh.at[pl.ds(s * 625 + 560, 65)],
                    r0_v.at[pl.ds(0, 65)])
    pltpu.sync_copy(r0_v.at[pl.ds(0, 65)],
                    out_hbm.at[c, pl.ds(s * 625 + 560, 65)])


def _wstack_body(comp_ref, bases_ref, w_ref):
    r = pl.program_id(0)
    acc = jnp.zeros((D, D), f32)
    for b in range(NB + 1):
        acc = acc + comp_ref[r, b] * bases_ref[b]
    w_ref[0] = acc


def _tc_wstack(comp_ext, bases_ext):
    return pl.pallas_call(
        _wstack_body,
        out_shape=jax.ShapeDtypeStruct((R + 1, D, D), f32),
        grid=(R + 1,),
        in_specs=[
            pl.BlockSpec(memory_space=pltpu.SMEM),
            pl.BlockSpec((NB + 1, D, D), lambda r: (0, 0, 0)),
        ],
        out_specs=pl.BlockSpec((1, D, D), lambda r: (r, 0, 0)),
    )(comp_ext, bases_ext)


def _ymm_body(h_ref, w_ref, y_ref):
    y_ref[0] = jnp.dot(h_ref[...], w_ref[0], preferred_element_type=f32)


def _tc_ymm(h, wstack):
    return pl.pallas_call(
        _ymm_body,
        out_shape=jax.ShapeDtypeStruct((R + 1, N, D), f32),
        grid=(NT, R + 1),
        in_specs=[
            pl.BlockSpec((BN, D), lambda i, r: (i, 0)),
            pl.BlockSpec((1, D, D), lambda i, r: (r, 0, 0)),
        ],
        out_specs=pl.BlockSpec((1, BN, D), lambda i, r: (r, i, 0)),
    )(h, wstack)


def _postA_body(p_ref, y_ref, bias_ref, z_ref, st_ref, acc_ref):
    i = pl.program_id(0)
    @pl.when(i == 0)
    def _():
        acc_ref[...] = jnp.zeros((8, D), f32)
    z = p_ref[0] + p_ref[1] + y_ref[0] + bias_ref[...]
    z = jnp.maximum(z, 0.0)
    z_ref[...] = z
    acc_ref[0:1, :] = acc_ref[0:1, :] + jnp.sum(z, axis=0, keepdims=True)
    acc_ref[1:2, :] = acc_ref[1:2, :] + jnp.sum(z * z, axis=0, keepdims=True)
    @pl.when(i == NT - 1)
    def _():
        st_ref[...] = acc_ref[...]


def _tc_postA(p, y, bias2):
    return pl.pallas_call(
        _postA_body,
        out_shape=[jax.ShapeDtypeStruct((N, D), f32),
                   jax.ShapeDtypeStruct((8, D), f32)],
        grid=(NT,),
        in_specs=[
            pl.BlockSpec((NC, BN, D), lambda i: (0, i, 0)),
            pl.BlockSpec((1, BN, D), lambda i: (R, i, 0)),
            pl.BlockSpec((1, D), lambda i: (0, 0)),
        ],
        out_specs=[pl.BlockSpec((BN, D), lambda i: (i, 0)),
                   pl.BlockSpec((8, D), lambda i: (0, 0))],
        scratch_shapes=[pltpu.VMEM((8, D), f32)],
    )(p, y, bias2)


def _postB_body_res(z_ref, st_ref, g_ref, b_ref, res_ref, h_ref):
    mu = st_ref[0:1, :] / N
    var = st_ref[1:2, :] / N - mu * mu
    rstd = lax.rsqrt(var + 1e-5)
    h = (z_ref[...] - mu) * rstd * g_ref[...] + b_ref[...]
    h_ref[...] = h + res_ref[...]


def _postB_body(z_ref, st_ref, g_ref, b_ref, h_ref):
    mu = st_ref[0:1, :] / N
    var = st_ref[1:2, :] / N - mu * mu
    rstd = lax.rsqrt(var + 1e-5)
    h_ref[...] = (z_ref[...] - mu) * rstd * g_ref[...] + b_ref[...]


def _tc_postB(z, stats, g2, b2, res=None):
    vec = pl.BlockSpec((1, D), lambda i: (0, 0))
    blk = pl.BlockSpec((BN, D), lambda i: (i, 0))
    st = pl.BlockSpec((8, D), lambda i: (0, 0))
    if res is None:
        return pl.pallas_call(
            _postB_body,
            out_shape=jax.ShapeDtypeStruct((N, D), f32),
            grid=(NT,),
            in_specs=[blk, st, vec, vec],
            out_specs=blk,
        )(z, stats, g2, b2)
    return pl.pallas_call(
        _postB_body_res,
        out_shape=jax.ShapeDtypeStruct((N, D), f32),
        grid=(NT,),
        in_specs=[blk, st, vec, vec, blk],
        out_specs=blk,
    )(z, stats, g2, b2, res)


def _post3_body(p_ref, y_ref, bias_ref, o_ref):
    o_ref[...] = p_ref[0] + p_ref[1] + y_ref[0] + bias_ref[...]


def _tc_post3(p, y, bias2):
    return pl.pallas_call(
        _post3_body,
        out_shape=jax.ShapeDtypeStruct((N, D), f32),
        grid=(NT,),
        in_specs=[
            pl.BlockSpec((NC, BN, D), lambda i: (0, i, 0)),
            pl.BlockSpec((1, BN, D), lambda i: (R, i, 0)),
            pl.BlockSpec((1, D), lambda i: (0, 0)),
        ],
        out_specs=pl.BlockSpec((BN, D), lambda i: (i, 0)),
    )(p, y, bias2)


def _conv_layer(h, gidx3, dst3, w2, comp, bases, root):
    comp_ext = jnp.zeros((R + 1, NB + 1), f32)
    comp_ext = comp_ext.at[:R, :NB].set(comp).at[R, NB].set(1.0)
    bases_ext = jnp.concatenate([bases, root[None]], axis=0)
    wstack = _tc_wstack(comp_ext, bases_ext)
    y = _tc_ymm(h, wstack)
    p = _sc_edge(y.reshape((R + 1) * N, D), gidx3, dst3, w2)
    return p, y


def kernel(x, edge_index, edge_type,
           comp1, bases1, root1, bias1,
           comp2, bases2, root2, bias2,
           comp3, bases3, root3, bias3,
           g1, b1, g2, b2):
    src = edge_index[0]
    dst = edge_index[1]
    et = edge_type

    parts = _sc_count(dst, et)
    winv = _tc_cinv(parts)
    gidx, w_rep = _sc_prep(src, dst, et, winv.reshape(RN, 16))
    gidx3 = gidx.reshape(NW, CE, KE)
    dst3 = dst.reshape(NW, CE, KE)
    w2 = w_rep

    p, y = _conv_layer(x, gidx3, dst3, w2, comp1, bases1, root1)
    z, st = _tc_postA(p, y, bias1[None, :])
    h1 = _tc_postB(z, st, g1[None, :], b1[None, :])
    p, y = _conv_layer(h1, gidx3, dst3, w2, comp2, bases2, root2)
    z, st = _tc_postA(p, y, bias2[None, :])
    h2 = _tc_postB(z, st, g2[None, :], b2[None, :], res=h1)
    p, y = _conv_layer(h2, gidx3, dst3, w2, comp3, bases3, root3)
    return _tc_post3(p, y, bias3[None, :])
